# baseline (speedup 1.0000x reference)
import functools

import jax
import jax.numpy as jnp
from jax import lax
from jax.experimental import pallas as pl
from jax.experimental.pallas import tpu as pltpu
from jax.experimental.pallas import tpu_sc as plsc

RANK = 64
KPAD = 1024
TILE = 256

_NC = 2
_NS = 16
_NW = _NC * _NS


_CH = 128
_PADW = 2 * RANK


def _make_sc_gather(total_rows: int):
    bpw = total_rows // _NW
    nch = bpw // _CH
    mesh = plsc.VectorSubcoreMesh(core_axis_name="c", subcore_axis_name="s")

    @functools.partial(
        pl.kernel,
        mesh=mesh,
        out_type=jax.ShapeDtypeStruct((total_rows, _PADW), jnp.float32),
        scratch_types=[
            pltpu.VMEM((nch, _CH), jnp.int32),
            pltpu.VMEM((_CH, _PADW), jnp.float32),
            pltpu.VMEM((_CH, _PADW), jnp.float32),
            pltpu.SemaphoreType.DMA,
            pltpu.SemaphoreType.DMA,
        ],
    )
    def gather(table_hbm, idx_hbm, out_hbm, idx_v, buf0, buf1, sem0, sem1):
        wid = lax.axis_index("s") * _NC + lax.axis_index("c")
        base = wid * bpw
        pltpu.sync_copy(idx_hbm.at[wid], idx_v)
        bufs = (buf0, buf1)
        sems = (sem0, sem1)
        copies = [None, None]
        for j in range(nch):
            copies[j % 2] = pltpu.async_copy(
                table_hbm.at[idx_v.at[j]], bufs[j % 2], sems[j % 2])
            if j >= 1:
                copies[(j - 1) % 2].wait()
                pltpu.sync_copy(bufs[(j - 1) % 2],
                                out_hbm.at[pl.ds(base + (j - 1) * _CH, _CH)])
        copies[(nch - 1) % 2].wait()
        pltpu.sync_copy(bufs[(nch - 1) % 2],
                        out_hbm.at[pl.ds(base + (nch - 1) * _CH, _CH)])

    return gather


def _normalize(x):
    norm = jnp.sqrt(jnp.sum(x * x, axis=1, keepdims=True))
    return x / jnp.maximum(norm, 1e-12)


def _tc_body(rel_ref, se_ref, oe_ref, nse_ref, noe_ref,
             sp_ref, op_ref, nsp_ref, nop_ref, rhi_ref,
             pexp_ref, qexp_ref,
             pos_ref, neg_ref, pregul_ref, nregul_ref, acc_ref):
    i = pl.program_id(0)
    nt = pl.num_programs(0)

    @pl.when(i == 0)
    def _init():
        acc_ref[0] = 0.0
        acc_ref[1] = 0.0
        acc_ref[2] = 0.0
        acc_ref[3] = 0.0
        acc_ref[4] = 0.0
        pregul_ref[...] = jnp.zeros_like(pregul_ref)
        nregul_ref[...] = jnp.zeros_like(nregul_ref)

    rel = rel_ref[...]
    iota = lax.broadcasted_iota(jnp.int32, (TILE, KPAD), 1)
    onehot = (rel == iota).astype(jnp.bfloat16)

    g = lax.dot_general(onehot, rhi_ref[...],
                        (((1,), (0,)), ((), ())),
                        preferred_element_type=jnp.float32)

    def pick_half(row_ref, par_ref):
        rows = row_ref[0]
        par = par_ref[0]
        return jnp.where(par == 1, rows[:, RANK:], rows[:, :RANK])

    se = pick_half(se_ref, sp_ref)
    oe = pick_half(oe_ref, op_ref)
    nse = pick_half(nse_ref, nsp_ref)
    noe = pick_half(noe_ref, nop_ref)

    s_n = _normalize(se)
    o_n = _normalize(oe)
    ns_n = _normalize(nse)
    no_n = _normalize(noe)

    def expand(x, pat_ref):
        return lax.dot_general(x.astype(jnp.bfloat16), pat_ref[...],
                               (((1,), (0,)), ((), ())),
                               preferred_element_type=jnp.float32)

    s_rep = expand(s_n, pexp_ref)
    o_rep = expand(o_n, qexp_ref)
    ns_rep = expand(ns_n, pexp_ref)
    no_rep = expand(no_n, qexp_ref)

    pos_ref[0] = jnp.sum(g * s_rep * o_rep, axis=1, keepdims=True)
    neg_ref[0] = jnp.sum(g * ns_rep * no_rep, axis=1, keepdims=True)

    acc_ref[0] += jnp.sum(se * se)
    acc_ref[1] += jnp.sum(oe * oe)
    acc_ref[2] += jnp.sum(nse * nse)
    acc_ref[3] += jnp.sum(noe * noe)
    acc_ref[4] += jnp.sum(g * g)

    @pl.when(i == nt - 1)
    def _finalize():
        b_total = nt * TILE
        ent_scale = 1.0 / (b_total * RANK)
        rel_scale = 1.0 / (b_total * RANK * RANK)
        rel_term = acc_ref[4] * rel_scale
        pregul_ref[...] = jnp.full(pregul_ref.shape, (
            acc_ref[0] * ent_scale + acc_ref[1] * ent_scale + rel_term) / 3.0,
            jnp.float32)
        nregul_ref[...] = jnp.full(nregul_ref.shape, (
            acc_ref[2] * ent_scale + acc_ref[3] * ent_scale + rel_term) / 3.0,
            jnp.float32)


def kernel(sub, obj, n_sub, n_obj, rel, ent_embedding, rel_embedding):
    b = sub.shape[0]
    nt = b // TILE
    total = 4 * b

    idx_all = jnp.concatenate([sub, obj, n_sub, n_obj]).astype(jnp.int32)
    idx3 = (idx_all // 2).reshape(_NW, total // (_NW * _CH), _CH)
    p4 = (idx_all % 2).reshape(4, b, 1)
    table2 = ent_embedding.reshape(-1, _PADW)
    gathered = _make_sc_gather(total)(table2, idx3)
    g4 = gathered.reshape(4, b, _PADW)

    rhi = jnp.pad(
        rel_embedding.reshape(-1, RANK * RANK).astype(jnp.bfloat16),
        ((0, KPAD - rel_embedding.shape[0]), (0, 0)))
    rel2 = rel.astype(jnp.int32).reshape(b, 1)

    lane = jnp.arange(RANK * RANK, dtype=jnp.int32).reshape(1, -1)
    col = jnp.arange(RANK, dtype=jnp.int32).reshape(-1, 1)
    pexp = (lane // RANK == col).astype(jnp.bfloat16)
    qexp = (lane % RANK == col).astype(jnp.bfloat16)

    ent_spec = lambda s: pl.BlockSpec((1, TILE, _PADW), lambda i: (s, i, 0))
    par_spec = lambda s: pl.BlockSpec((1, TILE, 1), lambda i: (s, i, 0))
    pos3, neg3, pregul, nregul = pl.pallas_call(
        _tc_body,
        grid=(nt,),
        in_specs=[
            pl.BlockSpec((TILE, 1), lambda i: (i, 0)),
            ent_spec(0), ent_spec(1), ent_spec(2), ent_spec(3),
            par_spec(0), par_spec(1), par_spec(2), par_spec(3),
            pl.BlockSpec((KPAD, RANK * RANK), lambda i: (0, 0)),
            pl.BlockSpec((RANK, RANK * RANK), lambda i: (0, 0)),
            pl.BlockSpec((RANK, RANK * RANK), lambda i: (0, 0)),
        ],
        out_specs=[
            pl.BlockSpec((1, TILE, 1), lambda i: (i, 0, 0)),
            pl.BlockSpec((1, TILE, 1), lambda i: (i, 0, 0)),
            pl.BlockSpec((1, 128), lambda i: (0, 0)),
            pl.BlockSpec((1, 128), lambda i: (0, 0)),
        ],
        out_shape=[
            jax.ShapeDtypeStruct((nt, TILE, 1), jnp.float32),
            jax.ShapeDtypeStruct((nt, TILE, 1), jnp.float32),
            jax.ShapeDtypeStruct((1, 128), jnp.float32),
            jax.ShapeDtypeStruct((1, 128), jnp.float32),
        ],
        scratch_shapes=[pltpu.SMEM((8,), jnp.float32)],
        compiler_params=pltpu.CompilerParams(
            dimension_semantics=("arbitrary",)),
    )(rel2, g4, g4, g4, g4, p4, p4, p4, p4, rhi, pexp, qexp)

    return (pos3.reshape(b), neg3.reshape(b),
            pregul[0, 0].reshape(()), nregul[0, 0].reshape(()))


# speedup vs baseline: 1.1148x; 1.1148x over previous
import functools

import jax
import jax.numpy as jnp
from jax import lax
from jax.experimental import pallas as pl
from jax.experimental.pallas import tpu as pltpu
from jax.experimental.pallas import tpu_sc as plsc

RANK = 64
KPAD = 1024
TILE = 256

_NC = 2
_NS = 16
_NW = _NC * _NS


_CH = 128
_PADW = 2 * RANK


def _make_sc_gather(total_rows: int):
    bpw = total_rows // _NW
    nch = bpw // _CH
    mesh = plsc.VectorSubcoreMesh(core_axis_name="c", subcore_axis_name="s")

    @functools.partial(
        pl.kernel,
        mesh=mesh,
        out_type=jax.ShapeDtypeStruct((total_rows, _PADW), jnp.float32),
        scratch_types=[
            pltpu.VMEM((nch, _CH), jnp.int32),
            pltpu.VMEM((_CH, _PADW), jnp.float32),
            pltpu.VMEM((_CH, _PADW), jnp.float32),
            pltpu.SemaphoreType.DMA,
            pltpu.SemaphoreType.DMA,
        ],
    )
    def gather(table_hbm, idx_hbm, out_hbm, idx_v, buf0, buf1, sem0, sem1):
        wid = lax.axis_index("s") * _NC + lax.axis_index("c")
        base = wid * bpw
        pltpu.sync_copy(idx_hbm.at[wid], idx_v)
        bufs = (buf0, buf1)
        sems = (sem0, sem1)
        copies = [None, None]
        for j in range(nch):
            copies[j % 2] = pltpu.async_copy(
                table_hbm.at[idx_v.at[j]], bufs[j % 2], sems[j % 2])
            if j >= 1:
                copies[(j - 1) % 2].wait()
                pltpu.sync_copy(bufs[(j - 1) % 2],
                                out_hbm.at[pl.ds(base + (j - 1) * _CH, _CH)])
        copies[(nch - 1) % 2].wait()
        pltpu.sync_copy(bufs[(nch - 1) % 2],
                        out_hbm.at[pl.ds(base + (nch - 1) * _CH, _CH)])

    return gather


def _normalize(x):
    norm = jnp.sqrt(jnp.sum(x * x, axis=1, keepdims=True))
    return x / jnp.maximum(norm, 1e-12)


def _tc_body(rel_ref, se_ref, oe_ref, nse_ref, noe_ref, rhi_ref,
             pexp_ref, qexp_ref,
             pos_ref, neg_ref, pregul_ref, nregul_ref, acc_ref):
    i = pl.program_id(0)
    nt = pl.num_programs(0)

    @pl.when(i == 0)
    def _init():
        acc_ref[0] = 0.0
        acc_ref[1] = 0.0
        acc_ref[2] = 0.0
        acc_ref[3] = 0.0
        acc_ref[4] = 0.0
        pregul_ref[...] = jnp.zeros_like(pregul_ref)
        nregul_ref[...] = jnp.zeros_like(nregul_ref)

    rel = rel_ref[...]
    iota = lax.broadcasted_iota(jnp.int32, (TILE, KPAD), 1)
    onehot = (rel == iota).astype(jnp.bfloat16)

    g = lax.dot_general(onehot, rhi_ref[...],
                        (((1,), (0,)), ((), ())),
                        preferred_element_type=jnp.float32)

    se = se_ref[0][:, :RANK]
    oe = oe_ref[0][:, :RANK]
    nse = nse_ref[0][:, :RANK]
    noe = noe_ref[0][:, :RANK]

    s_n = _normalize(se)
    o_n = _normalize(oe)
    ns_n = _normalize(nse)
    no_n = _normalize(noe)

    def expand(x, pat_ref):
        return lax.dot_general(x.astype(jnp.bfloat16), pat_ref[...],
                               (((1,), (0,)), ((), ())),
                               preferred_element_type=jnp.float32)

    s_rep = expand(s_n, pexp_ref)
    o_rep = expand(o_n, qexp_ref)
    ns_rep = expand(ns_n, pexp_ref)
    no_rep = expand(no_n, qexp_ref)

    pos_ref[0] = jnp.sum(g * s_rep * o_rep, axis=1, keepdims=True)
    neg_ref[0] = jnp.sum(g * ns_rep * no_rep, axis=1, keepdims=True)

    acc_ref[0] += jnp.sum(se * se)
    acc_ref[1] += jnp.sum(oe * oe)
    acc_ref[2] += jnp.sum(nse * nse)
    acc_ref[3] += jnp.sum(noe * noe)
    acc_ref[4] += jnp.sum(g * g)

    @pl.when(i == nt - 1)
    def _finalize():
        b_total = nt * TILE
        ent_scale = 1.0 / (b_total * RANK)
        rel_scale = 1.0 / (b_total * RANK * RANK)
        rel_term = acc_ref[4] * rel_scale
        pregul_ref[...] = jnp.full(pregul_ref.shape, (
            acc_ref[0] * ent_scale + acc_ref[1] * ent_scale + rel_term) / 3.0,
            jnp.float32)
        nregul_ref[...] = jnp.full(nregul_ref.shape, (
            acc_ref[2] * ent_scale + acc_ref[3] * ent_scale + rel_term) / 3.0,
            jnp.float32)


def kernel(sub, obj, n_sub, n_obj, rel, ent_embedding, rel_embedding):
    b = sub.shape[0]
    nt = b // TILE
    total = 4 * b

    idx_all = jnp.concatenate([sub, obj, n_sub, n_obj]).astype(jnp.int32)
    idx3 = idx_all.reshape(_NW, total // (_NW * _CH), _CH)
    table_pad = jnp.pad(ent_embedding, ((0, 0), (0, _PADW - RANK)))
    gathered = _make_sc_gather(total)(table_pad, idx3)
    g4 = gathered.reshape(4, b, _PADW)

    rhi = jnp.pad(
        rel_embedding.reshape(-1, RANK * RANK).astype(jnp.bfloat16),
        ((0, KPAD - rel_embedding.shape[0]), (0, 0)))
    rel2 = rel.astype(jnp.int32).reshape(b, 1)

    lane = jnp.arange(RANK * RANK, dtype=jnp.int32).reshape(1, -1)
    col = jnp.arange(RANK, dtype=jnp.int32).reshape(-1, 1)
    pexp = (lane // RANK == col).astype(jnp.bfloat16)
    qexp = (lane % RANK == col).astype(jnp.bfloat16)

    ent_spec = lambda s: pl.BlockSpec((1, TILE, _PADW), lambda i: (s, i, 0))
    pos3, neg3, pregul, nregul = pl.pallas_call(
        _tc_body,
        grid=(nt,),
        in_specs=[
            pl.BlockSpec((TILE, 1), lambda i: (i, 0)),
            ent_spec(0), ent_spec(1), ent_spec(2), ent_spec(3),
            pl.BlockSpec((KPAD, RANK * RANK), lambda i: (0, 0)),
            pl.BlockSpec((RANK, RANK * RANK), lambda i: (0, 0)),
            pl.BlockSpec((RANK, RANK * RANK), lambda i: (0, 0)),
        ],
        out_specs=[
            pl.BlockSpec((1, TILE, 1), lambda i: (i, 0, 0)),
            pl.BlockSpec((1, TILE, 1), lambda i: (i, 0, 0)),
            pl.BlockSpec((1, 128), lambda i: (0, 0)),
            pl.BlockSpec((1, 128), lambda i: (0, 0)),
        ],
        out_shape=[
            jax.ShapeDtypeStruct((nt, TILE, 1), jnp.float32),
            jax.ShapeDtypeStruct((nt, TILE, 1), jnp.float32),
            jax.ShapeDtypeStruct((1, 128), jnp.float32),
            jax.ShapeDtypeStruct((1, 128), jnp.float32),
        ],
        scratch_shapes=[pltpu.SMEM((8,), jnp.float32)],
        compiler_params=pltpu.CompilerParams(
            dimension_semantics=("arbitrary",)),
    )(rel2, g4, g4, g4, g4, rhi, pexp, qexp)

    return (pos3.reshape(b), neg3.reshape(b),
            pregul[0, 0].reshape(()), nregul[0, 0].reshape(()))
